# baseline (speedup 1.0000x reference)
import functools

import jax
import jax.numpy as jnp
from jax import lax
from jax.experimental import pallas as pl
from jax.experimental.pallas import tpu as pltpu
from jax.experimental.pallas import tpu_sc as plsc

_NC = 2
_NS = 16
_NW = _NC * _NS
_LANES = 16
_CHUNK = 128


def _gather_kernel(n_batch, table_rows, n_idx, d):
    nb = n_batch // _NW
    n_chunks = n_idx // _CHUNK

    mesh = plsc.VectorSubcoreMesh(
        core_axis_name="c", subcore_axis_name="s",
        num_cores=_NC, num_subcores=_NS)

    @functools.partial(
        pl.kernel,
        out_type=jax.ShapeDtypeStruct((n_batch * n_idx, d), jnp.float32),
        mesh=mesh,
        compiler_params=pltpu.CompilerParams(use_tc_tiling_on_sc=False),
        scratch_types=[
            pltpu.VMEM((n_idx,), jnp.int32),
            pltpu.VMEM((2, n_idx), jnp.int32),
            pltpu.VMEM((2, n_idx, d), jnp.float32),
            pltpu.SemaphoreType.DMA,
            pltpu.SemaphoreType.DMA,
            pltpu.SemaphoreType.DMA,
            pltpu.SemaphoreType.DMA,
        ],
    )
    def k(table_hbm, idx_hbm, out_hbm, idx0_v, idxb_v, rows_v,
          gsem0, gsem1, osem0, osem1):
        gsem = (gsem0, gsem1)
        osem = (osem0, osem1)
        wid = lax.axis_index("s") * _NC + lax.axis_index("c")
        base = wid * nb

        pltpu.sync_copy(idx_hbm, idx0_v)

        def stage_idx(s, b):
            off = b * table_rows
            def add16(j, _):
                sl = pl.ds(j * _LANES, _LANES)
                idxb_v[s, sl] = idx0_v[sl] + off
                return 0
            lax.fori_loop(0, n_idx // _LANES, add16, 0)

        def fire_gathers(s):
            for c in range(n_chunks):
                sl = pl.ds(c * _CHUNK, _CHUNK)
                pltpu.async_copy(
                    table_hbm.at[idxb_v.at[s, sl]],
                    rows_v.at[s, sl], gsem[s])

        def drain_gathers(s):
            for c in range(n_chunks):
                sl = pl.ds(c * _CHUNK, _CHUNK)
                pltpu.make_async_copy(
                    table_hbm.at[idxb_v.at[s, sl]],
                    rows_v.at[s, sl], gsem[s]).wait()

        def fire_scatter(s, b):
            pltpu.async_copy(rows_v.at[s],
                             out_hbm.at[pl.ds(b * n_idx, n_idx)], osem[s])

        def wait_scatter(s, b):
            pltpu.make_async_copy(
                rows_v.at[s],
                out_hbm.at[pl.ds(b * n_idx, n_idx)], osem[s]).wait()

        for s in range(2):
            stage_idx(s, base + s)
            fire_gathers(s)

        def outer(t, _):
            for s in range(2):
                b = base + 2 * t + s
                drain_gathers(s)
                fire_scatter(s, b)
                stage_idx(s, b + 2)
                wait_scatter(s, b)
                fire_gathers(s)
            return 0
        lax.fori_loop(0, (nb - 2) // 2, outer, 0)

        for s in range(2):
            b = base + nb - 2 + s
            drain_gathers(s)
            fire_scatter(s, b)
        for s in range(2):
            wait_scatter(s, base + nb - 2 + s)

    return k


def kernel(params, idx):
    n_batch, table_rows, d = params.shape
    i = idx[0, :, 0].astype(jnp.int32)
    n_idx = i.shape[0]
    flat = params.reshape(n_batch * table_rows, d)
    out = _gather_kernel(n_batch, table_rows, n_idx, d)(flat, i)
    return out.reshape(n_batch, n_idx, d)


# speedup vs baseline: 1.0013x; 1.0013x over previous
import functools

import jax
import jax.numpy as jnp
from jax import lax
from jax.experimental import pallas as pl
from jax.experimental.pallas import tpu as pltpu
from jax.experimental.pallas import tpu_sc as plsc

_NC = 2
_NS = 16
_NW = _NC * _NS
_LANES = 16
_CHUNK = 2048


def _gather_kernel(n_batch, table_rows, n_idx, d):
    nb = n_batch // _NW
    n_chunks = n_idx // _CHUNK

    mesh = plsc.VectorSubcoreMesh(
        core_axis_name="c", subcore_axis_name="s",
        num_cores=_NC, num_subcores=_NS)

    @functools.partial(
        pl.kernel,
        out_type=jax.ShapeDtypeStruct((n_batch * n_idx, d), jnp.float32),
        mesh=mesh,
        compiler_params=pltpu.CompilerParams(use_tc_tiling_on_sc=False),
        scratch_types=[
            pltpu.VMEM((n_idx,), jnp.int32),
            pltpu.VMEM((2, n_idx), jnp.int32),
            pltpu.VMEM((2, n_idx, d), jnp.float32),
            pltpu.SemaphoreType.DMA,
            pltpu.SemaphoreType.DMA,
            pltpu.SemaphoreType.DMA,
            pltpu.SemaphoreType.DMA,
        ],
    )
    def k(table_hbm, idx_hbm, out_hbm, idx0_v, idxb_v, rows_v,
          gsem0, gsem1, osem0, osem1):
        gsem = (gsem0, gsem1)
        osem = (osem0, osem1)
        wid = lax.axis_index("s") * _NC + lax.axis_index("c")
        base = wid * nb

        pltpu.sync_copy(idx_hbm, idx0_v)

        def stage_idx(s, b):
            off = b * table_rows
            def add16(j, _):
                sl = pl.ds(j * _LANES, _LANES)
                idxb_v[s, sl] = idx0_v[sl] + off
                return 0
            lax.fori_loop(0, n_idx // _LANES, add16, 0)

        def fire_gathers(s):
            for c in range(n_chunks):
                sl = pl.ds(c * _CHUNK, _CHUNK)
                pltpu.async_copy(
                    table_hbm.at[idxb_v.at[s, sl]],
                    rows_v.at[s, sl], gsem[s])

        def drain_gathers(s):
            for c in range(n_chunks):
                sl = pl.ds(c * _CHUNK, _CHUNK)
                pltpu.make_async_copy(
                    table_hbm.at[idxb_v.at[s, sl]],
                    rows_v.at[s, sl], gsem[s]).wait()

        def fire_scatter(s, b):
            pltpu.async_copy(rows_v.at[s],
                             out_hbm.at[pl.ds(b * n_idx, n_idx)], osem[s])

        def wait_scatter(s, b):
            pltpu.make_async_copy(
                rows_v.at[s],
                out_hbm.at[pl.ds(b * n_idx, n_idx)], osem[s]).wait()

        for s in range(2):
            stage_idx(s, base + s)
            fire_gathers(s)

        def outer(t, _):
            for s in range(2):
                b = base + 2 * t + s
                drain_gathers(s)
                fire_scatter(s, b)
                stage_idx(s, b + 2)
                wait_scatter(s, b)
                fire_gathers(s)
            return 0
        lax.fori_loop(0, (nb - 2) // 2, outer, 0)

        for s in range(2):
            b = base + nb - 2 + s
            drain_gathers(s)
            fire_scatter(s, b)
        for s in range(2):
            wait_scatter(s, base + nb - 2 + s)

    return k


def kernel(params, idx):
    n_batch, table_rows, d = params.shape
    i = idx[0, :, 0].astype(jnp.int32)
    n_idx = i.shape[0]
    flat = params.reshape(n_batch * table_rows, d)
    out = _gather_kernel(n_batch, table_rows, n_idx, d)(flat, i)
    return out.reshape(n_batch, n_idx, d)


# speedup vs baseline: 1.0208x; 1.0194x over previous
import functools

import jax
import jax.numpy as jnp
from jax import lax
from jax.experimental import pallas as pl
from jax.experimental.pallas import tpu as pltpu
from jax.experimental.pallas import tpu_sc as plsc

_NC = 2
_NS = 16
_NW = _NC * _NS


def _gather_kernel(n_batch, table_rows, n_idx, d):
    per_w = n_idx // _NW

    mesh = plsc.VectorSubcoreMesh(
        core_axis_name="c", subcore_axis_name="s",
        num_cores=_NC, num_subcores=_NS)

    @functools.partial(
        pl.kernel,
        out_type=jax.ShapeDtypeStruct((n_batch, n_idx, d), jnp.float32),
        mesh=mesh,
        scratch_types=[
            pltpu.VMEM((per_w,), jnp.int32),
            pltpu.VMEM((2, 256, d), jnp.float32),
            pltpu.SemaphoreType.DMA,
            pltpu.SemaphoreType.DMA,
            pltpu.SemaphoreType.DMA,
            pltpu.SemaphoreType.DMA,
            pltpu.SemaphoreType.DMA,
        ],
    )
    def k(table_hbm, idx_hbm, out_hbm, idx_v, buf_v,
          isem, gsem0, gsem1, osem0, osem1):
        gsem = (gsem0, gsem1)
        osem = (osem0, osem1)
        wid = lax.axis_index("s") * _NC + lax.axis_index("c")
        base = wid * per_w

        pltpu.async_copy(idx_hbm.at[pl.ds(base, per_w)], idx_v, isem).wait()

        cb_n = n_batch // 256

        def run_pair(v, j, cb0):
            bsl0 = pl.ds(cb0 * 256, 256)
            bsl1 = pl.ds((cb0 + 1) * 256, 256)
            pltpu.async_copy(table_hbm.at[bsl0, v, :], buf_v.at[0], gsem[0])
            pltpu.async_copy(table_hbm.at[bsl1, v, :], buf_v.at[1], gsem[1])
            pltpu.make_async_copy(
                table_hbm.at[bsl0, v, :], buf_v.at[0], gsem[0]).wait()
            pltpu.async_copy(buf_v.at[0], out_hbm.at[bsl0, j, :], osem[0])
            pltpu.make_async_copy(
                table_hbm.at[bsl1, v, :], buf_v.at[1], gsem[1]).wait()
            pltpu.async_copy(buf_v.at[1], out_hbm.at[bsl1, j, :], osem[1])
            pltpu.make_async_copy(
                buf_v.at[0], out_hbm.at[bsl0, j, :], osem[0]).wait()
            pltpu.make_async_copy(
                buf_v.at[1], out_hbm.at[bsl1, j, :], osem[1]).wait()

        def group(g, _):
            ivec = idx_v[pl.ds(g * 16, 16)]
            for lane in range(16):
                v = ivec[lane]
                j = base + g * 16 + lane
                def pair(cb2, _):
                    run_pair(v, j, 2 * cb2)
                    return 0
                lax.fori_loop(0, cb_n // 2, pair, 0)
            return 0
        lax.fori_loop(0, per_w // 16, group, 0)

    return k


def kernel(params, idx):
    n_batch, table_rows, d = params.shape
    i = idx[0, :, 0].astype(jnp.int32)
    n_idx = i.shape[0]
    out = _gather_kernel(n_batch, table_rows, n_idx, d)(params, i)
    return out


# speedup vs baseline: 2.2321x; 2.1867x over previous
import functools

import jax
import jax.numpy as jnp
from jax import lax
from jax.experimental import pallas as pl
from jax.experimental.pallas import tpu as pltpu
from jax.experimental.pallas import tpu_sc as plsc

_NC = 2
_NS = 16
_NW = _NC * _NS


def _gather_kernel(n_batch, table_rows, n_idx, d):
    per_t = n_batch // _NW
    half = n_idx // 2
    rpw = 128 // d
    srows = table_rows // rpw
    orows = half // rpw

    mesh = plsc.VectorSubcoreMesh(
        core_axis_name="c", subcore_axis_name="s",
        num_cores=_NC, num_subcores=_NS)

    @functools.partial(
        pl.kernel,
        out_type=jax.ShapeDtypeStruct((n_batch, 2 * orows, 128), jnp.float32),
        mesh=mesh,
        scratch_types=[
            pltpu.VMEM((srows, 128), jnp.float32),
            pltpu.VMEM((n_idx,), jnp.int32),
            pltpu.VMEM((2, orows, 128), jnp.float32),
            pltpu.SemaphoreType.DMA,
            pltpu.SemaphoreType.DMA,
            pltpu.SemaphoreType.DMA,
            pltpu.SemaphoreType.DMA,
        ],
    )
    def k(table_hbm, idx_hbm, out_hbm, slab_v, idx_v, rows_v,
          isem, ssem, osem0, osem1):
        osem = (osem0, osem1)
        wid = lax.axis_index("s") * _NC + lax.axis_index("c")
        b0 = wid * per_t

        pltpu.async_copy(idx_hbm, idx_v, isem).wait()

        def fire_slab(b):
            pltpu.async_copy(table_hbm.at[b0 + b], slab_v, ssem)

        def wait_slab(b):
            pltpu.make_async_copy(table_hbm.at[b0 + b], slab_v, ssem).wait()

        def gather_half(h):
            def grp(q, _):
                ivec = idx_v[pl.ds((h * half // 16 + q) * 16, 16)]
                for lane in range(16):
                    v = ivec[lane]
                    src = slab_v[v // rpw, pl.ds((v % rpw) * d, d)]
                    jj = q * 16 + lane
                    rows_v[h, jj // rpw, pl.ds((jj % rpw) * d, d)] = src
                return 0
            lax.fori_loop(0, half // 16, grp, 0)

        def fire_scat(h, b):
            pltpu.async_copy(
                rows_v.at[h],
                out_hbm.at[b0 + b, pl.ds(h * orows, orows), :], osem[h])

        def wait_scat(h, b):
            pltpu.make_async_copy(
                rows_v.at[h],
                out_hbm.at[b0 + b, pl.ds(h * orows, orows), :], osem[h]).wait()

        fire_slab(0)
        wait_slab(0)
        for h in range(2):
            gather_half(h)
            fire_scat(h, 0)
        fire_slab(1)

        def body(t, _):
            b = 1 + t
            wait_slab(b)
            for h in range(2):
                wait_scat(h, b - 1)
                gather_half(h)
                fire_scat(h, b)
            fire_slab(b + 1)
            return 0
        lax.fori_loop(0, per_t - 2, body, 0)

        b_last = per_t - 1
        wait_slab(b_last)
        for h in range(2):
            wait_scat(h, b_last - 1)
            gather_half(h)
            fire_scat(h, b_last)
        for h in range(2):
            wait_scat(h, b_last)

    return k


def kernel(params, idx):
    n_batch, table_rows, d = params.shape
    i = idx[0, :, 0].astype(jnp.int32)
    n_idx = i.shape[0]
    table128 = params.reshape(n_batch, table_rows * d // 128, 128)
    out = _gather_kernel(n_batch, table_rows, n_idx, d)(table128, i)
    return out.reshape(n_batch, n_idx, d)
